# Initial kernel scaffold; baseline (speedup 1.0000x reference)
import jax
import jax.numpy as jnp
from jax.experimental import pallas as pl


def kernel(x, edge_index, batch, w1_root, w1_rel, b1, bn1_g, bn1_b, w2_root, w2_rel, b2, bn2_g, bn2_b, lw1, lb1, lw2, lb2):
    raise NotImplementedError("write your pallas kernel here")


# speedup vs baseline: 6.0313x; 6.0313x over previous
import functools

import jax
import jax.numpy as jnp
from jax import lax
from jax.experimental import pallas as pl
from jax.experimental.pallas import tpu as pltpu
from jax.experimental.pallas import tpu_sc as plsc

_NC = 2
_NS = 16
_NW = _NC * _NS
_K = 80
_W = 32


def _sc_segment_sum(y, src, dst, zeros):
  n = y.shape[0]
  e = src.shape[0]
  epw = e // _NW
  nb = epw // _K
  assert epw * _NW == e and nb * _K == epw and n % _NS == 0
  rpt = n // _NS

  mesh = plsc.VectorSubcoreMesh(
      core_axis_name="c", subcore_axis_name="s",
      num_cores=_NC, num_subcores=_NS)

  @functools.partial(
      pl.kernel,
      out_type=jax.ShapeDtypeStruct((_NC, n, _W), jnp.float32),
      mesh=mesh,
      scratch_types=[
          pltpu.VMEM((_K,), jnp.int32),
          pltpu.VMEM((_K,), jnp.int32),
          pltpu.VMEM((_K, _W), jnp.float32),
          pltpu.VMEM_SHARED((n, _W), jnp.float32),
          pltpu.SemaphoreType.DMA,
      ],
      compiler_params=pltpu.CompilerParams(use_tc_tiling_on_sc=False),
  )
  def body(y_hbm, src_hbm, dst_hbm, z_hbm, out_hbm, src_v, dst_v, rows_v,
           acc, sem):
    c = lax.axis_index("c")
    s = lax.axis_index("s")
    w = c * _NS + s
    pltpu.sync_copy(z_hbm.at[pl.ds(s * rpt, rpt)],
                    acc.at[pl.ds(s * rpt, rpt)])
    plsc.subcore_barrier()

    def step(i, carry):
      off = w * epw + i * _K
      pltpu.sync_copy(src_hbm.at[pl.ds(off, _K)], src_v)
      pltpu.sync_copy(dst_hbm.at[pl.ds(off, _K)], dst_v)
      pltpu.async_copy(y_hbm.at[src_v], rows_v, sem).wait()
      pltpu.sync_copy(rows_v, acc.at[dst_v], add=True)
      return carry

    lax.fori_loop(0, nb, step, 0)
    plsc.subcore_barrier()
    pltpu.sync_copy(acc.at[pl.ds(s * rpt, rpt)],
                    out_hbm.at[c, pl.ds(s * rpt, rpt)])

  return body(y, src, dst, zeros)


def _dot(a, b):
  return lax.dot_general(a, b, (((1,), (0,)), ((), ())),
                         preferred_element_type=jnp.float32)


def _dot_t(a, b):
  return lax.dot_general(a, b, (((0,), (0,)), ((), ())),
                         preferred_element_type=jnp.float32)


def _bn_relu(agg0, agg1, root, b, g, bb, n_real):
  rows = lax.broadcasted_iota(jnp.int32, (agg0.shape[0], 1), 0)
  m = (rows < n_real).astype(jnp.float32)
  u = jnp.maximum(agg0 + agg1 + root + b, 0.0) * m
  mu = jnp.sum(u, axis=0, keepdims=True) * (1.0 / n_real)
  d = (u - mu) * m
  var = jnp.sum(d * d, axis=0, keepdims=True) * (1.0 / n_real)
  return d * lax.rsqrt(var + 1e-5) * g + bb


def _head(h, batch, lw1, lb1, lw2, lb2, n_groups, width):
  onehot_mask = batch == lax.broadcasted_iota(jnp.int32, (1, n_groups), 1)
  onehot = onehot_mask.astype(jnp.float32)
  counts = jnp.sum(onehot, axis=0, keepdims=True)
  seg_sum_t = _dot_t(h, onehot)
  big = jnp.where(onehot_mask, 0.0, -1e30)
  lane = lax.broadcasted_iota(jnp.int32, (1, h.shape[1]), 1)
  row = lax.broadcasted_iota(jnp.int32, (width, 1), 0)

  def max_row(f, carry):
    col = jnp.sum(jnp.where(lane == f, h, 0.0), axis=1, keepdims=True)
    r = jnp.max(col + big, axis=0, keepdims=True)
    return jnp.where(row == f, r, carry)

  seg_max_t = lax.fori_loop(0, width,
                            max_row, jnp.zeros((width, n_groups),
                                               jnp.float32))
  seg_max_t = jnp.where(counts > 0, seg_max_t, 0.0)
  seg_mean_t = seg_sum_t[:width] / jnp.maximum(counts, 1.0)
  z_t = jnp.concatenate([seg_max_t, seg_mean_t], axis=0)
  z1_t = jnp.maximum(_dot_t(lw1, z_t) + lb1, 0.0)
  out_t = _dot_t(lw2, z1_t) + lb2
  return 1.0 / (1.0 + jnp.exp(-out_t))


def _tc_project(x, w_rel, w_root):
  def body(x_ref, wrel_ref, wroot_ref, y_ref, root_ref):
    xv = x_ref[...]
    y_ref[...] = _dot(xv, wrel_ref[...])
    root_ref[...] = _dot(xv, wroot_ref[...])

  n = x.shape[0]
  return pl.pallas_call(
      body,
      out_shape=[jax.ShapeDtypeStruct((n, _W), jnp.float32),
                 jax.ShapeDtypeStruct((n, _W), jnp.float32)],
  )(x, w_rel, w_root)


def _tc_mid(agg, root, b, g, bb, w2_rel, w2_root, n_real):
  def body(agg_ref, root_ref, b_ref, g_ref, bb_ref, wrel_ref, wroot_ref,
           y_ref, root2_ref):
    h = _bn_relu(agg_ref[0], agg_ref[1], root_ref[...], b_ref[...],
                 g_ref[...], bb_ref[...], n_real)
    y_ref[...] = _dot(h, wrel_ref[...])
    root2_ref[...] = _dot(h, wroot_ref[...])

  n = root.shape[0]
  return pl.pallas_call(
      body,
      out_shape=[jax.ShapeDtypeStruct((n, _W), jnp.float32),
                 jax.ShapeDtypeStruct((n, _W), jnp.float32)],
  )(agg, root, b, g, bb, w2_rel, w2_root)


def _tc_tail(agg, root, b, g, bb, batch, lw1, lb1, lw2, lb2, n_groups,
             n_real):
  def body(agg_ref, root_ref, b_ref, g_ref, bb_ref, batch_ref, lw1_ref,
           lb1_ref, lw2_ref, lb2_ref, out_ref):
    h = _bn_relu(agg_ref[0], agg_ref[1], root_ref[...], b_ref[...],
                 g_ref[...], bb_ref[...], n_real)
    out_ref[...] = _head(h, batch_ref[...], lw1_ref[...], lb1_ref[...],
                         lw2_ref[...], lb2_ref[...], n_groups, 20)

  return pl.pallas_call(
      body,
      out_shape=jax.ShapeDtypeStruct((1, n_groups), jnp.float32),
  )(agg, root, b, g, bb, batch, lw1, lb1, lw2, lb2)


def kernel(x, edge_index, batch, w1_root, w1_rel, b1, bn1_g, bn1_b,
           w2_root, w2_rel, b2, bn2_g, bn2_b, lw1, lb1, lw2, lb2):
  n = x.shape[0]
  n_groups = 64
  n_pad = ((n + _NS * 8 - 1) // (_NS * 8)) * (_NS * 8)

  def pad_w(w):
    return jnp.pad(w, ((0, _W - w.shape[0]), (0, _W - w.shape[1])))

  def pad_v(v, fill=0.0):
    return jnp.pad(v, (0, _W - v.shape[0]),
                   constant_values=fill).reshape(1, _W)

  w1_rel_p = jnp.pad(w1_rel, ((0, 0), (0, _W - w1_rel.shape[1])))
  w1_root_p = jnp.pad(w1_root, ((0, 0), (0, _W - w1_root.shape[1])))
  w2_rel_p = pad_w(w2_rel)
  w2_root_p = pad_w(w2_root)
  b1_p = pad_v(b1)
  g1_p = pad_v(bn1_g, 1.0)
  bb1_p = pad_v(bn1_b)
  b2_p = pad_v(b2)
  g2_p = pad_v(bn2_g, 1.0)
  bb2_p = pad_v(bn2_b)

  src = edge_index[0]
  dst = edge_index[1]
  zeros = jnp.zeros((n_pad, _W), jnp.float32)
  x_p = jnp.pad(x, ((0, n_pad - n), (0, 0)))
  batch2 = jnp.pad(batch, (0, n_pad - n),
                   constant_values=n_groups).reshape(n_pad, 1)
  lb1_c = lb1.reshape(-1, 1)
  lb2_c = lb2.reshape(-1, 1)

  y1, root1 = _tc_project(x_p, w1_rel_p, w1_root_p)
  agg1 = _sc_segment_sum(y1, src, dst, zeros)
  y2, root2 = _tc_mid(agg1, root1, b1_p, g1_p, bb1_p, w2_rel_p, w2_root_p, n)
  agg2 = _sc_segment_sum(y2, src, dst, zeros)
  out_t = _tc_tail(agg2, root2, b2_p, g2_p, bb2_p, batch2,
                   lw1, lb1_c, lw2, lb2_c, n_groups, n)
  return out_t.reshape(n_groups, 1)


# speedup vs baseline: 9.6881x; 1.6063x over previous
import functools

import jax
import jax.numpy as jnp
from jax import lax
from jax.experimental import pallas as pl
from jax.experimental.pallas import tpu as pltpu
from jax.experimental.pallas import tpu_sc as plsc

_NC = 2
_NS = 16
_NW = _NC * _NS
_K = 128
_NBUF = 4
_W = 32


def _sc_segment_sum(y, src2d, dst2d, zeros):
  n = y.shape[0]
  blocks = src2d.shape[0]
  nb = blocks // _NW
  assert nb * _NW == blocks and nb % _NBUF == 0 and n % (_NS * 8) == 0
  rpt = n // _NS
  outer = nb // _NBUF

  mesh = plsc.VectorSubcoreMesh(
      core_axis_name="c", subcore_axis_name="s",
      num_cores=_NC, num_subcores=_NS)

  @functools.partial(
      pl.kernel,
      out_type=jax.ShapeDtypeStruct((_NC, n, _W), jnp.float32),
      mesh=mesh,
      scratch_types=[
          pltpu.VMEM((nb, _K), jnp.int32),
          pltpu.VMEM((nb, _K), jnp.int32),
          [pltpu.VMEM((_K, _W), jnp.float32)] * _NBUF,
          [pltpu.SemaphoreType.DMA] * _NBUF,
          [pltpu.SemaphoreType.DMA] * _NBUF,
          pltpu.VMEM_SHARED((n, _W), jnp.float32),
      ],
      compiler_params=pltpu.CompilerParams(use_tc_tiling_on_sc=False),
  )
  def body(y_hbm, src_hbm, dst_hbm, z_hbm, out_hbm, src_v, dst_v, bufs,
           gsems, ssems, acc):
    c = lax.axis_index("c")
    s = lax.axis_index("s")
    w = c * _NS + s
    pltpu.sync_copy(src_hbm.at[pl.ds(w * nb, nb)], src_v)
    pltpu.sync_copy(dst_hbm.at[pl.ds(w * nb, nb)], dst_v)
    pltpu.sync_copy(z_hbm.at[pl.ds(s * rpt, rpt)],
                    acc.at[pl.ds(s * rpt, rpt)])
    plsc.subcore_barrier()

    def gather(blk, b):
      return pltpu.async_copy(y_hbm.at[src_v.at[blk]], bufs[b], gsems[b])

    def gather_wait(b):
      pltpu.make_async_copy(y_hbm.at[pl.ds(0, _K)], bufs[b], gsems[b]).wait()

    for b in range(_NBUF):
      gather(b, b)

    def step(j, carry):
      base = j * _NBUF
      scatters = []
      for b in range(_NBUF):
        gather_wait(b)
        scatters.append(pltpu.async_copy(
            bufs[b], acc.at[dst_v.at[base + b]], ssems[b], add=True))
      for b in range(_NBUF):
        scatters[b].wait()
        @pl.when(j < outer - 1)
        def _():
          gather(base + _NBUF + b, b)
      return carry

    lax.fori_loop(0, outer, step, 0)
    plsc.subcore_barrier()
    pltpu.sync_copy(acc.at[pl.ds(s * rpt, rpt)],
                    out_hbm.at[c, pl.ds(s * rpt, rpt)])

  return body(y, src2d, dst2d, zeros)


def _dot(a, b):
  return lax.dot_general(a, b, (((1,), (0,)), ((), ())),
                         preferred_element_type=jnp.float32)


def _dot_t(a, b):
  return lax.dot_general(a, b, (((0,), (0,)), ((), ())),
                         preferred_element_type=jnp.float32)


def _bn_relu(agg0, agg1, root, b, g, bb, n_real):
  rows = lax.broadcasted_iota(jnp.int32, (agg0.shape[0], 1), 0)
  m = (rows < n_real).astype(jnp.float32)
  u = jnp.maximum(agg0 + agg1 + root + b, 0.0) * m
  mu = jnp.sum(u, axis=0, keepdims=True) * (1.0 / n_real)
  d = (u - mu) * m
  var = jnp.sum(d * d, axis=0, keepdims=True) * (1.0 / n_real)
  return d * lax.rsqrt(var + 1e-5) * g + bb


def _head(h, batch, lw1, lb1, lw2, lb2, n_groups, width):
  onehot_mask = batch == lax.broadcasted_iota(jnp.int32, (1, n_groups), 1)
  onehot = onehot_mask.astype(jnp.float32)
  counts = jnp.sum(onehot, axis=0, keepdims=True)
  seg_sum_t = _dot_t(h, onehot)
  big = jnp.where(onehot_mask, 0.0, -1e30)
  lane = lax.broadcasted_iota(jnp.int32, (1, h.shape[1]), 1)
  row = lax.broadcasted_iota(jnp.int32, (width, 1), 0)

  def max_row(f, carry):
    col = jnp.sum(jnp.where(lane == f, h, 0.0), axis=1, keepdims=True)
    r = jnp.max(col + big, axis=0, keepdims=True)
    return jnp.where(row == f, r, carry)

  seg_max_t = lax.fori_loop(0, width,
                            max_row, jnp.zeros((width, n_groups),
                                               jnp.float32))
  seg_max_t = jnp.where(counts > 0, seg_max_t, 0.0)
  seg_mean_t = seg_sum_t[:width] / jnp.maximum(counts, 1.0)
  z_t = jnp.concatenate([seg_max_t, seg_mean_t], axis=0)
  z1_t = jnp.maximum(_dot_t(lw1, z_t) + lb1, 0.0)
  out_t = _dot_t(lw2, z1_t) + lb2
  return 1.0 / (1.0 + jnp.exp(-out_t))


def _tc_project(x, w_rel, w_root):
  def body(x_ref, wrel_ref, wroot_ref, y_ref, root_ref):
    xv = x_ref[...]
    y_ref[...] = _dot(xv, wrel_ref[...])
    root_ref[...] = _dot(xv, wroot_ref[...])

  n = x.shape[0]
  return pl.pallas_call(
      body,
      out_shape=[jax.ShapeDtypeStruct((n, _W), jnp.float32),
                 jax.ShapeDtypeStruct((n, _W), jnp.float32)],
  )(x, w_rel, w_root)


def _tc_mid(agg, root, b, g, bb, w2_rel, w2_root, n_real):
  def body(agg_ref, root_ref, b_ref, g_ref, bb_ref, wrel_ref, wroot_ref,
           y_ref, root2_ref):
    h = _bn_relu(agg_ref[0], agg_ref[1], root_ref[...], b_ref[...],
                 g_ref[...], bb_ref[...], n_real)
    y_ref[...] = _dot(h, wrel_ref[...])
    root2_ref[...] = _dot(h, wroot_ref[...])

  n = root.shape[0]
  return pl.pallas_call(
      body,
      out_shape=[jax.ShapeDtypeStruct((n, _W), jnp.float32),
                 jax.ShapeDtypeStruct((n, _W), jnp.float32)],
  )(agg, root, b, g, bb, w2_rel, w2_root)


def _tc_tail(agg, root, b, g, bb, batch, lw1, lb1, lw2, lb2, n_groups,
             n_real):
  def body(agg_ref, root_ref, b_ref, g_ref, bb_ref, batch_ref, lw1_ref,
           lb1_ref, lw2_ref, lb2_ref, out_ref):
    h = _bn_relu(agg_ref[0], agg_ref[1], root_ref[...], b_ref[...],
                 g_ref[...], bb_ref[...], n_real)
    out_ref[...] = _head(h, batch_ref[...], lw1_ref[...], lb1_ref[...],
                         lw2_ref[...], lb2_ref[...], n_groups, 20)

  return pl.pallas_call(
      body,
      out_shape=jax.ShapeDtypeStruct((1, n_groups), jnp.float32),
  )(agg, root, b, g, bb, batch, lw1, lb1, lw2, lb2)


def kernel(x, edge_index, batch, w1_root, w1_rel, b1, bn1_g, bn1_b,
           w2_root, w2_rel, b2, bn2_g, bn2_b, lw1, lb1, lw2, lb2):
  n = x.shape[0]
  n_groups = 64
  n_pad = ((n + _NS * 8 - 1) // (_NS * 8)) * (_NS * 8)

  def pad_w(w):
    return jnp.pad(w, ((0, _W - w.shape[0]), (0, _W - w.shape[1])))

  def pad_v(v, fill=0.0):
    return jnp.pad(v, (0, _W - v.shape[0]),
                   constant_values=fill).reshape(1, _W)

  w1_rel_p = jnp.pad(w1_rel, ((0, 0), (0, _W - w1_rel.shape[1])))
  w1_root_p = jnp.pad(w1_root, ((0, 0), (0, _W - w1_root.shape[1])))
  w2_rel_p = pad_w(w2_rel)
  w2_root_p = pad_w(w2_root)
  b1_p = pad_v(b1)
  g1_p = pad_v(bn1_g, 1.0)
  bb1_p = pad_v(bn1_b)
  b2_p = pad_v(b2)
  g2_p = pad_v(bn2_g, 1.0)
  bb2_p = pad_v(bn2_b)

  e = edge_index.shape[1]
  blk_per_tile = -(-e // (_NW * _K * _NBUF)) * _NBUF
  e_pad = blk_per_tile * _NW * _K
  src = jnp.pad(edge_index[0], (0, e_pad - e),
                constant_values=n_pad - 1).reshape(-1, _K)
  dst = jnp.pad(edge_index[1], (0, e_pad - e),
                constant_values=n_pad - 1).reshape(-1, _K)
  zeros = jnp.zeros((n_pad, _W), jnp.float32)
  x_p = jnp.pad(x, ((0, n_pad - n), (0, 0)))
  batch2 = jnp.pad(batch, (0, n_pad - n),
                   constant_values=n_groups).reshape(n_pad, 1)
  lb1_c = lb1.reshape(-1, 1)
  lb2_c = lb2.reshape(-1, 1)

  y1, root1 = _tc_project(x_p, w1_rel_p, w1_root_p)
  agg1 = _sc_segment_sum(y1, src, dst, zeros)
  y2, root2 = _tc_mid(agg1, root1, b1_p, g1_p, bb1_p, w2_rel_p, w2_root_p, n)
  agg2 = _sc_segment_sum(y2, src, dst, zeros)
  out_t = _tc_tail(agg2, root2, b2_p, g2_p, bb2_p, batch2,
                   lw1, lb1_c, lw2, lb2_c, n_groups, n)
  return out_t.reshape(n_groups, 1)


# speedup vs baseline: 9.9635x; 1.0284x over previous
import functools

import jax
import jax.numpy as jnp
from jax import lax
from jax.experimental import pallas as pl
from jax.experimental.pallas import tpu as pltpu
from jax.experimental.pallas import tpu_sc as plsc

_NC = 2
_NS = 16
_NW = _NC * _NS
_K = 128
_NBUF = 8
_W = 32


def _sc_segment_sum(y, src2d, dst2d, zeros):
  n = y.shape[0]
  blocks = src2d.shape[0]
  nb = blocks // _NW
  assert nb * _NW == blocks and nb % _NBUF == 0 and n % (_NS * 8) == 0
  rpt = n // _NS
  outer = nb // _NBUF

  mesh = plsc.VectorSubcoreMesh(
      core_axis_name="c", subcore_axis_name="s",
      num_cores=_NC, num_subcores=_NS)

  @functools.partial(
      pl.kernel,
      out_type=jax.ShapeDtypeStruct((_NC, n, _W), jnp.float32),
      mesh=mesh,
      scratch_types=[
          pltpu.VMEM((nb, _K), jnp.int32),
          pltpu.VMEM((nb, _K), jnp.int32),
          [pltpu.VMEM((_K, _W), jnp.float32)] * _NBUF,
          [pltpu.SemaphoreType.DMA] * _NBUF,
          [pltpu.SemaphoreType.DMA] * _NBUF,
          pltpu.VMEM_SHARED((n, _W), jnp.float32),
      ],
      compiler_params=pltpu.CompilerParams(use_tc_tiling_on_sc=False),
  )
  def body(y_hbm, src_hbm, dst_hbm, z_hbm, out_hbm, src_v, dst_v, bufs,
           gsems, ssems, acc):
    c = lax.axis_index("c")
    s = lax.axis_index("s")
    w = c * _NS + s
    pltpu.sync_copy(src_hbm.at[pl.ds(w * nb, nb)], src_v)
    pltpu.sync_copy(dst_hbm.at[pl.ds(w * nb, nb)], dst_v)
    pltpu.sync_copy(z_hbm.at[pl.ds(s * rpt, rpt)],
                    acc.at[pl.ds(s * rpt, rpt)])
    plsc.subcore_barrier()

    def gather(blk, b):
      return pltpu.async_copy(y_hbm.at[src_v.at[blk]], bufs[b], gsems[b])

    def gather_wait(b):
      pltpu.make_async_copy(y_hbm.at[pl.ds(0, _K)], bufs[b], gsems[b]).wait()

    for b in range(_NBUF):
      gather(b, b)

    def step(j, carry):
      base = j * _NBUF
      scatters = []
      for b in range(_NBUF):
        gather_wait(b)
        scatters.append(pltpu.async_copy(
            bufs[b], acc.at[dst_v.at[base + b]], ssems[b], add=True))
      for b in range(_NBUF):
        scatters[b].wait()
        @pl.when(j < outer - 1)
        def _():
          gather(base + _NBUF + b, b)
      return carry

    lax.fori_loop(0, outer, step, 0)
    plsc.subcore_barrier()
    pltpu.sync_copy(acc.at[pl.ds(s * rpt, rpt)],
                    out_hbm.at[c, pl.ds(s * rpt, rpt)])

  return body(y, src2d, dst2d, zeros)


def _dot(a, b):
  return lax.dot_general(a, b, (((1,), (0,)), ((), ())),
                         preferred_element_type=jnp.float32)


def _dot_t(a, b):
  return lax.dot_general(a, b, (((0,), (0,)), ((), ())),
                         preferred_element_type=jnp.float32)


def _bn_relu(agg0, agg1, root, b, g, bb, n_real):
  rows = lax.broadcasted_iota(jnp.int32, (agg0.shape[0], 1), 0)
  m = (rows < n_real).astype(jnp.float32)
  u = jnp.maximum(agg0 + agg1 + root + b, 0.0) * m
  mu = jnp.sum(u, axis=0, keepdims=True) * (1.0 / n_real)
  d = (u - mu) * m
  var = jnp.sum(d * d, axis=0, keepdims=True) * (1.0 / n_real)
  return d * lax.rsqrt(var + 1e-5) * g + bb


def _head(h, batch, lw1, lb1, lw2, lb2, n_groups, width):
  onehot_mask = batch == lax.broadcasted_iota(jnp.int32, (1, n_groups), 1)
  onehot = onehot_mask.astype(jnp.float32)
  counts = jnp.sum(onehot, axis=0, keepdims=True)
  seg_sum_t = _dot_t(h, onehot)
  big = jnp.where(onehot_mask, 0.0, -1e30)
  lane = lax.broadcasted_iota(jnp.int32, (1, h.shape[1]), 1)
  row = lax.broadcasted_iota(jnp.int32, (width, 1), 0)

  def max_row(f, carry):
    col = jnp.sum(jnp.where(lane == f, h, 0.0), axis=1, keepdims=True)
    r = jnp.max(col + big, axis=0, keepdims=True)
    return jnp.where(row == f, r, carry)

  seg_max_t = lax.fori_loop(0, width,
                            max_row, jnp.zeros((width, n_groups),
                                               jnp.float32))
  seg_max_t = jnp.where(counts > 0, seg_max_t, 0.0)
  seg_mean_t = seg_sum_t[:width] / jnp.maximum(counts, 1.0)
  z_t = jnp.concatenate([seg_max_t, seg_mean_t], axis=0)
  z1_t = jnp.maximum(_dot_t(lw1, z_t) + lb1, 0.0)
  out_t = _dot_t(lw2, z1_t) + lb2
  return 1.0 / (1.0 + jnp.exp(-out_t))


def _tc_project(x, w_rel, w_root):
  def body(x_ref, wrel_ref, wroot_ref, y_ref, root_ref):
    xv = x_ref[...]
    y_ref[...] = _dot(xv, wrel_ref[...])
    root_ref[...] = _dot(xv, wroot_ref[...])

  n = x.shape[0]
  return pl.pallas_call(
      body,
      out_shape=[jax.ShapeDtypeStruct((n, _W), jnp.float32),
                 jax.ShapeDtypeStruct((n, _W), jnp.float32)],
  )(x, w_rel, w_root)


def _tc_mid(agg, root, b, g, bb, w2_rel, w2_root, n_real):
  def body(agg_ref, root_ref, b_ref, g_ref, bb_ref, wrel_ref, wroot_ref,
           y_ref, root2_ref):
    h = _bn_relu(agg_ref[0], agg_ref[1], root_ref[...], b_ref[...],
                 g_ref[...], bb_ref[...], n_real)
    y_ref[...] = _dot(h, wrel_ref[...])
    root2_ref[...] = _dot(h, wroot_ref[...])

  n = root.shape[0]
  return pl.pallas_call(
      body,
      out_shape=[jax.ShapeDtypeStruct((n, _W), jnp.float32),
                 jax.ShapeDtypeStruct((n, _W), jnp.float32)],
  )(agg, root, b, g, bb, w2_rel, w2_root)


def _tc_tail(agg, root, b, g, bb, batch, lw1, lb1, lw2, lb2, n_groups,
             n_real):
  def body(agg_ref, root_ref, b_ref, g_ref, bb_ref, batch_ref, lw1_ref,
           lb1_ref, lw2_ref, lb2_ref, out_ref):
    h = _bn_relu(agg_ref[0], agg_ref[1], root_ref[...], b_ref[...],
                 g_ref[...], bb_ref[...], n_real)
    out_ref[...] = _head(h, batch_ref[...], lw1_ref[...], lb1_ref[...],
                         lw2_ref[...], lb2_ref[...], n_groups, 20)

  return pl.pallas_call(
      body,
      out_shape=jax.ShapeDtypeStruct((1, n_groups), jnp.float32),
  )(agg, root, b, g, bb, batch, lw1, lb1, lw2, lb2)


def kernel(x, edge_index, batch, w1_root, w1_rel, b1, bn1_g, bn1_b,
           w2_root, w2_rel, b2, bn2_g, bn2_b, lw1, lb1, lw2, lb2):
  n = x.shape[0]
  n_groups = 64
  n_pad = ((n + _NS * 8 - 1) // (_NS * 8)) * (_NS * 8)

  def pad_w(w):
    return jnp.pad(w, ((0, _W - w.shape[0]), (0, _W - w.shape[1])))

  def pad_v(v, fill=0.0):
    return jnp.pad(v, (0, _W - v.shape[0]),
                   constant_values=fill).reshape(1, _W)

  w1_rel_p = jnp.pad(w1_rel, ((0, 0), (0, _W - w1_rel.shape[1])))
  w1_root_p = jnp.pad(w1_root, ((0, 0), (0, _W - w1_root.shape[1])))
  w2_rel_p = pad_w(w2_rel)
  w2_root_p = pad_w(w2_root)
  b1_p = pad_v(b1)
  g1_p = pad_v(bn1_g, 1.0)
  bb1_p = pad_v(bn1_b)
  b2_p = pad_v(b2)
  g2_p = pad_v(bn2_g, 1.0)
  bb2_p = pad_v(bn2_b)

  e = edge_index.shape[1]
  blk_per_tile = -(-e // (_NW * _K * _NBUF)) * _NBUF
  e_pad = blk_per_tile * _NW * _K
  src = jnp.pad(edge_index[0], (0, e_pad - e),
                constant_values=n_pad - 1).reshape(-1, _K)
  dst = jnp.pad(edge_index[1], (0, e_pad - e),
                constant_values=n_pad - 1).reshape(-1, _K)
  zeros = jnp.zeros((n_pad, _W), jnp.float32)
  x_p = jnp.pad(x, ((0, n_pad - n), (0, 0)))
  batch2 = jnp.pad(batch, (0, n_pad - n),
                   constant_values=n_groups).reshape(n_pad, 1)
  lb1_c = lb1.reshape(-1, 1)
  lb2_c = lb2.reshape(-1, 1)

  y1, root1 = _tc_project(x_p, w1_rel_p, w1_root_p)
  agg1 = _sc_segment_sum(y1, src, dst, zeros)
  y2, root2 = _tc_mid(agg1, root1, b1_p, g1_p, bb1_p, w2_rel_p, w2_root_p, n)
  agg2 = _sc_segment_sum(y2, src, dst, zeros)
  out_t = _tc_tail(agg2, root2, b2_p, g2_p, bb2_p, batch2,
                   lw1, lb1_c, lw2, lb2_c, n_groups, n)
  return out_t.reshape(n_groups, 1)


# speedup vs baseline: 15.1776x; 1.5233x over previous
import functools

import jax
import jax.numpy as jnp
from jax import lax
from jax.experimental import pallas as pl
from jax.experimental.pallas import tpu as pltpu
from jax.experimental.pallas import tpu_sc as plsc

_NC = 2
_NS = 16
_NW = _NC * _NS
_K = 128
_NBUF = 8
_W = 32


def _sc_segment_sum(y, src2d, dst2d, zeros):
  n = y.shape[0]
  blocks = src2d.shape[0]
  nb = blocks // _NW
  assert nb * _NW == blocks and nb % _NBUF == 0 and n % (_NS * 8) == 0
  rpt = n // _NS
  outer = nb // _NBUF

  mesh = plsc.VectorSubcoreMesh(
      core_axis_name="c", subcore_axis_name="s",
      num_cores=_NC, num_subcores=_NS)

  @functools.partial(
      pl.kernel,
      out_type=jax.ShapeDtypeStruct((_NC, n, _W), jnp.float32),
      mesh=mesh,
      scratch_types=[
          pltpu.VMEM((nb, _K), jnp.int32),
          pltpu.VMEM((nb, _K), jnp.int32),
          [pltpu.VMEM((_K, _W), jnp.float32)] * _NBUF,
          [pltpu.SemaphoreType.DMA] * _NBUF,
          [pltpu.SemaphoreType.DMA] * _NBUF,
          pltpu.VMEM_SHARED((n, _W), jnp.float32),
          pltpu.VMEM_SHARED((n, _W), jnp.float32),
      ],
      compiler_params=pltpu.CompilerParams(use_tc_tiling_on_sc=False),
  )
  def body(y_hbm, src_hbm, dst_hbm, z_hbm, out_hbm, src_v, dst_v, bufs,
           gsems, ssems, acc, y_sh):
    c = lax.axis_index("c")
    s = lax.axis_index("s")
    w = c * _NS + s
    pltpu.sync_copy(src_hbm.at[pl.ds(w * nb, nb)], src_v)
    pltpu.sync_copy(dst_hbm.at[pl.ds(w * nb, nb)], dst_v)
    pltpu.sync_copy(y_hbm.at[pl.ds(s * rpt, rpt)],
                    y_sh.at[pl.ds(s * rpt, rpt)])
    pltpu.sync_copy(z_hbm.at[pl.ds(s * rpt, rpt)],
                    acc.at[pl.ds(s * rpt, rpt)])
    plsc.subcore_barrier()

    def gather(blk, b):
      return pltpu.async_copy(y_sh.at[src_v.at[blk]], bufs[b], gsems[b])

    def gather_wait(b):
      pltpu.make_async_copy(y_hbm.at[pl.ds(0, _K)], bufs[b], gsems[b]).wait()

    for b in range(_NBUF):
      gather(b, b)

    def step(j, carry):
      base = j * _NBUF
      scatters = []
      for b in range(_NBUF):
        gather_wait(b)
        scatters.append(pltpu.async_copy(
            bufs[b], acc.at[dst_v.at[base + b]], ssems[b], add=True))
      for b in range(_NBUF):
        scatters[b].wait()
        @pl.when(j < outer - 1)
        def _():
          gather(base + _NBUF + b, b)
      return carry

    lax.fori_loop(0, outer, step, 0)
    plsc.subcore_barrier()
    pltpu.sync_copy(acc.at[pl.ds(s * rpt, rpt)],
                    out_hbm.at[c, pl.ds(s * rpt, rpt)])

  return body(y, src2d, dst2d, zeros)


def _dot(a, b):
  return lax.dot_general(a, b, (((1,), (0,)), ((), ())),
                         preferred_element_type=jnp.float32)


def _dot_t(a, b):
  return lax.dot_general(a, b, (((0,), (0,)), ((), ())),
                         preferred_element_type=jnp.float32)


def _bn_relu(agg0, agg1, root, b, g, bb, n_real):
  rows = lax.broadcasted_iota(jnp.int32, (agg0.shape[0], 1), 0)
  m = (rows < n_real).astype(jnp.float32)
  u = jnp.maximum(agg0 + agg1 + root + b, 0.0) * m
  mu = jnp.sum(u, axis=0, keepdims=True) * (1.0 / n_real)
  d = (u - mu) * m
  var = jnp.sum(d * d, axis=0, keepdims=True) * (1.0 / n_real)
  return d * lax.rsqrt(var + 1e-5) * g + bb


def _head(h, batch, lw1, lb1, lw2, lb2, n_groups, width):
  onehot_mask = batch == lax.broadcasted_iota(jnp.int32, (1, n_groups), 1)
  onehot = onehot_mask.astype(jnp.float32)
  counts = jnp.sum(onehot, axis=0, keepdims=True)
  seg_sum_t = _dot_t(h, onehot)
  big = jnp.where(onehot_mask, 0.0, -1e30)
  lane = lax.broadcasted_iota(jnp.int32, (1, h.shape[1]), 1)
  row = lax.broadcasted_iota(jnp.int32, (width, 1), 0)

  def max_row(f, carry):
    col = jnp.sum(jnp.where(lane == f, h, 0.0), axis=1, keepdims=True)
    r = jnp.max(col + big, axis=0, keepdims=True)
    return jnp.where(row == f, r, carry)

  seg_max_t = lax.fori_loop(0, width,
                            max_row, jnp.zeros((width, n_groups),
                                               jnp.float32))
  seg_max_t = jnp.where(counts > 0, seg_max_t, 0.0)
  seg_mean_t = seg_sum_t[:width] / jnp.maximum(counts, 1.0)
  z_t = jnp.concatenate([seg_max_t, seg_mean_t], axis=0)
  z1_t = jnp.maximum(_dot_t(lw1, z_t) + lb1, 0.0)
  out_t = _dot_t(lw2, z1_t) + lb2
  return 1.0 / (1.0 + jnp.exp(-out_t))


def _tc_project(x, w_rel, w_root):
  def body(x_ref, wrel_ref, wroot_ref, y_ref, root_ref):
    xv = x_ref[...]
    y_ref[...] = _dot(xv, wrel_ref[...])
    root_ref[...] = _dot(xv, wroot_ref[...])

  n = x.shape[0]
  return pl.pallas_call(
      body,
      out_shape=[jax.ShapeDtypeStruct((n, _W), jnp.float32),
                 jax.ShapeDtypeStruct((n, _W), jnp.float32)],
  )(x, w_rel, w_root)


def _tc_mid(agg, root, b, g, bb, w2_rel, w2_root, n_real):
  def body(agg_ref, root_ref, b_ref, g_ref, bb_ref, wrel_ref, wroot_ref,
           y_ref, root2_ref):
    h = _bn_relu(agg_ref[0], agg_ref[1], root_ref[...], b_ref[...],
                 g_ref[...], bb_ref[...], n_real)
    y_ref[...] = _dot(h, wrel_ref[...])
    root2_ref[...] = _dot(h, wroot_ref[...])

  n = root.shape[0]
  return pl.pallas_call(
      body,
      out_shape=[jax.ShapeDtypeStruct((n, _W), jnp.float32),
                 jax.ShapeDtypeStruct((n, _W), jnp.float32)],
  )(agg, root, b, g, bb, w2_rel, w2_root)


def _tc_tail(agg, root, b, g, bb, batch, lw1, lb1, lw2, lb2, n_groups,
             n_real):
  def body(agg_ref, root_ref, b_ref, g_ref, bb_ref, batch_ref, lw1_ref,
           lb1_ref, lw2_ref, lb2_ref, out_ref):
    h = _bn_relu(agg_ref[0], agg_ref[1], root_ref[...], b_ref[...],
                 g_ref[...], bb_ref[...], n_real)
    out_ref[...] = _head(h, batch_ref[...], lw1_ref[...], lb1_ref[...],
                         lw2_ref[...], lb2_ref[...], n_groups, 20)

  return pl.pallas_call(
      body,
      out_shape=jax.ShapeDtypeStruct((1, n_groups), jnp.float32),
  )(agg, root, b, g, bb, batch, lw1, lb1, lw2, lb2)


def kernel(x, edge_index, batch, w1_root, w1_rel, b1, bn1_g, bn1_b,
           w2_root, w2_rel, b2, bn2_g, bn2_b, lw1, lb1, lw2, lb2):
  n = x.shape[0]
  n_groups = 64
  n_pad = ((n + _NS * 8 - 1) // (_NS * 8)) * (_NS * 8)

  def pad_w(w):
    return jnp.pad(w, ((0, _W - w.shape[0]), (0, _W - w.shape[1])))

  def pad_v(v, fill=0.0):
    return jnp.pad(v, (0, _W - v.shape[0]),
                   constant_values=fill).reshape(1, _W)

  w1_rel_p = jnp.pad(w1_rel, ((0, 0), (0, _W - w1_rel.shape[1])))
  w1_root_p = jnp.pad(w1_root, ((0, 0), (0, _W - w1_root.shape[1])))
  w2_rel_p = pad_w(w2_rel)
  w2_root_p = pad_w(w2_root)
  b1_p = pad_v(b1)
  g1_p = pad_v(bn1_g, 1.0)
  bb1_p = pad_v(bn1_b)
  b2_p = pad_v(b2)
  g2_p = pad_v(bn2_g, 1.0)
  bb2_p = pad_v(bn2_b)

  e = edge_index.shape[1]
  blk_per_tile = -(-e // (_NW * _K * _NBUF)) * _NBUF
  e_pad = blk_per_tile * _NW * _K
  src = jnp.pad(edge_index[0], (0, e_pad - e),
                constant_values=n_pad - 1).reshape(-1, _K)
  dst = jnp.pad(edge_index[1], (0, e_pad - e),
                constant_values=n_pad - 1).reshape(-1, _K)
  zeros = jnp.zeros((n_pad, _W), jnp.float32)
  x_p = jnp.pad(x, ((0, n_pad - n), (0, 0)))
  batch2 = jnp.pad(batch, (0, n_pad - n),
                   constant_values=n_groups).reshape(n_pad, 1)
  lb1_c = lb1.reshape(-1, 1)
  lb2_c = lb2.reshape(-1, 1)

  y1, root1 = _tc_project(x_p, w1_rel_p, w1_root_p)
  agg1 = _sc_segment_sum(y1, src, dst, zeros)
  y2, root2 = _tc_mid(agg1, root1, b1_p, g1_p, bb1_p, w2_rel_p, w2_root_p, n)
  agg2 = _sc_segment_sum(y2, src, dst, zeros)
  out_t = _tc_tail(agg2, root2, b2_p, g2_p, bb2_p, batch2,
                   lw1, lb1_c, lw2, lb2_c, n_groups, n)
  return out_t.reshape(n_groups, 1)


# speedup vs baseline: 16.3868x; 1.0797x over previous
import functools

import jax
import jax.numpy as jnp
from jax import lax
from jax.experimental import pallas as pl
from jax.experimental.pallas import tpu as pltpu
from jax.experimental.pallas import tpu_sc as plsc

_NC = 2
_NS = 16
_NW = _NC * _NS
_K = 128
_NBUF = 8
_W = 32


def _sc_segment_sum(y, src2d, dst2d, zeros):
  n = y.shape[0]
  blocks = src2d.shape[0]
  nb = blocks // _NW
  assert nb * _NW == blocks and nb % _NBUF == 0 and n % (_NS * 8) == 0
  rpt = n // _NS
  outer = nb // _NBUF

  mesh = plsc.VectorSubcoreMesh(
      core_axis_name="c", subcore_axis_name="s",
      num_cores=_NC, num_subcores=_NS)

  @functools.partial(
      pl.kernel,
      out_type=jax.ShapeDtypeStruct((_NC, n, _W), jnp.float32),
      mesh=mesh,
      scratch_types=[
          pltpu.VMEM((nb, _K), jnp.int32),
          pltpu.VMEM((nb, _K), jnp.int32),
          [pltpu.VMEM((_K, _W), jnp.float32)] * _NBUF,
          [pltpu.SemaphoreType.DMA] * _NBUF,
          [pltpu.SemaphoreType.DMA] * _NBUF,
          pltpu.VMEM_SHARED((n, _W), jnp.float32),
          pltpu.VMEM_SHARED((n, _W), jnp.float32),
      ],
      compiler_params=pltpu.CompilerParams(use_tc_tiling_on_sc=False),
  )
  def body(y_hbm, src_hbm, dst_hbm, z_hbm, out_hbm, src_v, dst_v, bufs,
           gsems, ssems, acc, y_sh):
    c = lax.axis_index("c")
    s = lax.axis_index("s")
    w = c * _NS + s
    pltpu.sync_copy(src_hbm.at[pl.ds(w * nb, nb)], src_v)
    pltpu.sync_copy(dst_hbm.at[pl.ds(w * nb, nb)], dst_v)
    pltpu.sync_copy(y_hbm.at[pl.ds(s * rpt, rpt)],
                    y_sh.at[pl.ds(s * rpt, rpt)])
    pltpu.sync_copy(z_hbm.at[pl.ds(s * rpt, rpt)],
                    acc.at[pl.ds(s * rpt, rpt)])
    plsc.subcore_barrier()

    def gather(blk, b):
      return pltpu.async_copy(y_sh.at[src_v.at[blk]], bufs[b], gsems[b])

    def gather_wait(b):
      pltpu.make_async_copy(y_hbm.at[pl.ds(0, _K)], bufs[b], gsems[b]).wait()

    for b in range(_NBUF):
      gather(b, b)

    def step(j, carry):
      base = j * _NBUF
      scatters = []
      for b in range(_NBUF):
        gather_wait(b)
        scatters.append(pltpu.async_copy(
            bufs[b], acc.at[dst_v.at[base + b]], ssems[b], add=True))
      for b in range(_NBUF):
        scatters[b].wait()
        @pl.when(j < outer - 1)
        def _():
          gather(base + _NBUF + b, b)
      return carry

    lax.fori_loop(0, outer, step, 0)
    plsc.subcore_barrier()
    pltpu.sync_copy(acc.at[pl.ds(s * rpt, rpt)],
                    out_hbm.at[c, pl.ds(s * rpt, rpt)])

  return body(y, src2d, dst2d, zeros)


def _dot(a, b):
  return lax.dot_general(a, b, (((1,), (0,)), ((), ())),
                         preferred_element_type=jnp.float32)


def _dot_t(a, b):
  return lax.dot_general(a, b, (((0,), (0,)), ((), ())),
                         preferred_element_type=jnp.float32)


def _bn_relu(agg0, agg1, root, b, g, bb, n_real):
  rows = lax.broadcasted_iota(jnp.int32, (agg0.shape[0], 1), 0)
  m = (rows < n_real).astype(jnp.float32)
  u = jnp.maximum(agg0 + agg1 + root + b, 0.0) * m
  mu = jnp.sum(u, axis=0, keepdims=True) * (1.0 / n_real)
  d = (u - mu) * m
  var = jnp.sum(d * d, axis=0, keepdims=True) * (1.0 / n_real)
  return d * lax.rsqrt(var + 1e-5) * g + bb


def _head(h, batch_col, batch_row, lw1, lb1, lw2, lb2, n_groups, width):
  w_pad = h.shape[1]
  onehot = (batch_col == lax.broadcasted_iota(
      jnp.int32, (1, n_groups), 1)).astype(jnp.float32)
  seg_sum = _dot_t(onehot, h)
  big_t = jnp.where(
      batch_row == lax.broadcasted_iota(jnp.int32, (n_groups, 1), 0),
      0.0, -1e30)
  counts = jnp.sum(jnp.where(big_t == 0.0, 1.0, 0.0),
                   axis=1, keepdims=True)
  h_t = lax.transpose(h, (1, 0))
  feat = lax.broadcasted_iota(jnp.int32, (w_pad, 1), 0)
  lane = lax.broadcasted_iota(jnp.int32, (1, w_pad), 1)

  def max_row(f, carry):
    row = jnp.sum(jnp.where(feat == f, h_t, 0.0), axis=0, keepdims=True)
    m = jnp.max(big_t + row, axis=1, keepdims=True)
    return jnp.where(lane == f, m, carry)

  seg_max = lax.fori_loop(0, width, max_row,
                          jnp.zeros((n_groups, w_pad), jnp.float32))
  seg_max = jnp.where(counts > 0, seg_max, 0.0)
  seg_mean = seg_sum / jnp.maximum(counts, 1.0)
  z = jnp.concatenate([seg_max[:, :width], seg_mean[:, :width]], axis=1)
  z1 = jnp.maximum(_dot(z, lw1) + lb1, 0.0)
  out = _dot(z1, lw2) + lb2
  return 1.0 / (1.0 + jnp.exp(-out))


def _tc_project(x, w_rel, w_root):
  def body(x_ref, wrel_ref, wroot_ref, y_ref, root_ref):
    xv = x_ref[...]
    y_ref[...] = _dot(xv, wrel_ref[...])
    root_ref[...] = _dot(xv, wroot_ref[...])

  n = x.shape[0]
  return pl.pallas_call(
      body,
      out_shape=[jax.ShapeDtypeStruct((n, _W), jnp.float32),
                 jax.ShapeDtypeStruct((n, _W), jnp.float32)],
  )(x, w_rel, w_root)


def _tc_mid(agg, root, b, g, bb, w2_rel, w2_root, n_real):
  def body(agg_ref, root_ref, b_ref, g_ref, bb_ref, wrel_ref, wroot_ref,
           y_ref, root2_ref):
    h = _bn_relu(agg_ref[0], agg_ref[1], root_ref[...], b_ref[...],
                 g_ref[...], bb_ref[...], n_real)
    y_ref[...] = _dot(h, wrel_ref[...])
    root2_ref[...] = _dot(h, wroot_ref[...])

  n = root.shape[0]
  return pl.pallas_call(
      body,
      out_shape=[jax.ShapeDtypeStruct((n, _W), jnp.float32),
                 jax.ShapeDtypeStruct((n, _W), jnp.float32)],
  )(agg, root, b, g, bb, w2_rel, w2_root)


def _tc_tail(agg, root, b, g, bb, batch_col, batch_row, lw1, lb1, lw2, lb2,
             n_groups, n_real):
  def body(agg_ref, root_ref, b_ref, g_ref, bb_ref, bcol_ref, brow_ref,
           lw1_ref, lb1_ref, lw2_ref, lb2_ref, out_ref):
    h = _bn_relu(agg_ref[0], agg_ref[1], root_ref[...], b_ref[...],
                 g_ref[...], bb_ref[...], n_real)
    out_ref[...] = _head(h, bcol_ref[...], brow_ref[...], lw1_ref[...],
                         lb1_ref[...], lw2_ref[...], lb2_ref[...],
                         n_groups, 20)

  return pl.pallas_call(
      body,
      out_shape=jax.ShapeDtypeStruct((n_groups, 1), jnp.float32),
  )(agg, root, b, g, bb, batch_col, batch_row, lw1, lb1, lw2, lb2)


def kernel(x, edge_index, batch, w1_root, w1_rel, b1, bn1_g, bn1_b,
           w2_root, w2_rel, b2, bn2_g, bn2_b, lw1, lb1, lw2, lb2):
  n = x.shape[0]
  n_groups = 64
  n_pad = ((n + _NS * 8 - 1) // (_NS * 8)) * (_NS * 8)

  def pad_w(w):
    return jnp.pad(w, ((0, _W - w.shape[0]), (0, _W - w.shape[1])))

  def pad_v(v, fill=0.0):
    return jnp.pad(v, (0, _W - v.shape[0]),
                   constant_values=fill).reshape(1, _W)

  w1_rel_p = jnp.pad(w1_rel, ((0, 0), (0, _W - w1_rel.shape[1])))
  w1_root_p = jnp.pad(w1_root, ((0, 0), (0, _W - w1_root.shape[1])))
  w2_rel_p = pad_w(w2_rel)
  w2_root_p = pad_w(w2_root)
  b1_p = pad_v(b1)
  g1_p = pad_v(bn1_g, 1.0)
  bb1_p = pad_v(bn1_b)
  b2_p = pad_v(b2)
  g2_p = pad_v(bn2_g, 1.0)
  bb2_p = pad_v(bn2_b)

  e = edge_index.shape[1]
  blk_per_tile = -(-e // (_NW * _K * _NBUF)) * _NBUF
  e_pad = blk_per_tile * _NW * _K
  src = jnp.pad(edge_index[0], (0, e_pad - e),
                constant_values=n_pad - 1).reshape(-1, _K)
  dst = jnp.pad(edge_index[1], (0, e_pad - e),
                constant_values=n_pad - 1).reshape(-1, _K)
  zeros = jnp.zeros((n_pad, _W), jnp.float32)
  x_p = jnp.pad(x, ((0, n_pad - n), (0, 0)))
  batch_p = jnp.pad(batch, (0, n_pad - n), constant_values=n_groups)
  batch_col = batch_p.reshape(n_pad, 1)
  batch_row = batch_p.reshape(1, n_pad)
  lb1_r = lb1.reshape(1, -1)
  lb2_r = lb2.reshape(1, -1)

  y1, root1 = _tc_project(x_p, w1_rel_p, w1_root_p)
  agg1 = _sc_segment_sum(y1, src, dst, zeros)
  y2, root2 = _tc_mid(agg1, root1, b1_p, g1_p, bb1_p, w2_rel_p, w2_root_p, n)
  agg2 = _sc_segment_sum(y2, src, dst, zeros)
  return _tc_tail(agg2, root2, b2_p, g2_p, bb2_p, batch_col, batch_row,
                  lw1, lb1_r, lw2, lb2_r, n_groups, n)
